# baseline (speedup 1.0000x reference)
import functools

import jax
import jax.numpy as jnp
from jax import lax
from jax.experimental import pallas as pl
from jax.experimental.pallas import tpu as pltpu
from jax.experimental.pallas import tpu_sc as plsc

N = 10000
F = 128
E = 320000
NC = 2
NS = 16
NW = NC * NS
B = 128
NCH = 80
EPW = B * NCH
EPAD = NW * EPW
NPAD = 10016
RINIT = 1000
RCOPY = 2000

SZ0 = (56, 56, 48)
SZ1 = (0, 0, 0)
K0 = sum(SZ0)
K1 = sum(SZ1)
KSTMAX = max(SZ0)
C1BASE = NS * K0
TOTCH = C1BASE
CAP_E = TOTCH * B

_mesh = plsc.VectorSubcoreMesh(core_axis_name="c", subcore_axis_name="s")


@functools.partial(
    pl.kernel,
    out_type=jax.ShapeDtypeStruct((NC, N, F), jnp.float32),
    mesh=_mesh,
    scratch_types=[
        pltpu.VMEM((NCH, B), jnp.int32),
        pltpu.VMEM((B, F), jnp.float32),
        pltpu.VMEM_SHARED((NPAD, F), jnp.float32),
    ],
)
def _sc_degree(col3, ones_hbm, zrows_hbm, dout, cidx, onesv, dacc):
    c = lax.axis_index("c")
    s = lax.axis_index("s")
    wid = s * NC + c
    pltpu.sync_copy(col3.at[wid], cidx)
    pltpu.sync_copy(ones_hbm, onesv)

    @pl.when(s < 10)
    def _():
        pltpu.sync_copy(zrows_hbm, dacc.at[pl.ds(s * RINIT, RINIT)])

    plsc.subcore_barrier()

    def body(j, carry):
        pltpu.sync_copy(onesv, dacc.at[cidx.at[j]], add=True)
        return carry

    lax.fori_loop(0, NCH, body, 0)
    plsc.subcore_barrier()

    @pl.when(s < 5)
    def _():
        pltpu.sync_copy(dacc.at[pl.ds(s * RCOPY, RCOPY)],
                        dout.at[c, pl.ds(s * RCOPY, RCOPY)])


@functools.partial(
    pl.kernel,
    out_type=jax.ShapeDtypeStruct((NC, N, F), jnp.float32),
    mesh=_mesh,
    scratch_types=[
        pltpu.VMEM((KSTMAX, B), jnp.int32),
        pltpu.VMEM((KSTMAX, B), jnp.int32),
        pltpu.VMEM((B, F), jnp.float32),
        pltpu.VMEM((B, F), jnp.float32),
        pltpu.VMEM_SHARED((NPAD, F), jnp.float32),
        pltpu.SemaphoreType.DMA,
        pltpu.SemaphoreType.DMA,
        pltpu.SemaphoreType.DMA,
        pltpu.SemaphoreType.DMA,
    ],
)
def _sc_agg(u_hbm, row2, col2, zrows_hbm, zout,
            ridx, cidx, rows0, rows1, acc, gs0, gs1, ss0, ss1):
    c = lax.axis_index("c")
    s = lax.axis_index("s")
    base = jnp.where(c == 0, s * K0, C1BASE + s * K1)

    @pl.when(s < 10)
    def _():
        pltpu.sync_copy(zrows_hbm, acc.at[pl.ds(s * RINIT, RINIT)])

    plsc.subcore_barrier()

    def _gather(j, rows, sem):
        return pltpu.async_copy(u_hbm.at[ridx.at[j]], rows, sem)

    def _gather_wait(j, rows, sem):
        pltpu.make_async_copy(u_hbm.at[ridx.at[j]], rows, sem).wait()

    def _scatter(j, rows, sem):
        return pltpu.async_copy(rows, acc.at[cidx.at[j]], sem, add=True)

    def _scatter_wait(j, rows, sem):
        pltpu.make_async_copy(rows, acc.at[cidx.at[j]], sem).wait()

    off0 = off1 = 0
    for st in range(len(SZ0)):
        start = pl.multiple_of(
            base + jnp.where(c == 0, off0, off1), 8)
        szc = jnp.where(c == 0, SZ0[st], SZ1[st])
        npair = jnp.where(c == 0, SZ0[st] // 2, SZ1[st] // 2)
        off0 += SZ0[st]
        off1 += SZ1[st]

        @pl.when(npair > 0)
        def _():
            pltpu.sync_copy(row2.at[pl.ds(start, KSTMAX)], ridx)
            pltpu.sync_copy(col2.at[pl.ds(start, KSTMAX)], cidx)
            _gather(0, rows0, gs0)
            _gather(1, rows1, gs1)

        def body(jp, carry):
            j0 = 2 * jp
            _gather_wait(j0, rows0, gs0)
            _scatter(j0, rows0, ss0)
            _gather_wait(j0 + 1, rows1, gs1)
            _scatter(j0 + 1, rows1, ss1)

            @pl.when(jp < npair - 1)
            def _():
                _scatter_wait(j0, rows0, ss0)
                _gather(j0 + 2, rows0, gs0)
                _scatter_wait(j0 + 1, rows1, ss1)
                _gather(j0 + 3, rows1, gs1)

            return carry

        lax.fori_loop(0, npair, body, 0)

        @pl.when(npair > 0)
        def _():
            _scatter_wait(szc - 2, rows0, ss0)
            _scatter_wait(szc - 1, rows1, ss1)

    plsc.subcore_barrier()

    @pl.when(s < 5)
    def _():
        pltpu.sync_copy(acc.at[pl.ds(s * RCOPY, RCOPY)],
                        zout.at[c, pl.ds(s * RCOPY, RCOPY)])


_BM = 2000


def _dinv_block(d0_ref, d1_ref):
    deg = d0_ref[:, 0:1] + d1_ref[:, 0:1] + 1.0
    return lax.rsqrt(deg)


def _tc_first_body(x_ref, w_ref, d0_ref, d1_ref, o_ref):
    h = jnp.dot(x_ref[...], w_ref[...], preferred_element_type=jnp.float32)
    o_ref[...] = h * _dinv_block(d0_ref, d1_ref)


def _tc_mid_body(z0_ref, z1_ref, u_ref, d0_ref, d1_ref, b_ref, w_ref, o_ref):
    dinv = _dinv_block(d0_ref, d1_ref)
    t = (z0_ref[...] + z1_ref[...] + u_ref[...]) * dinv + b_ref[...]
    t = jnp.maximum(t, 0.0)
    h = jnp.dot(t, w_ref[...], preferred_element_type=jnp.float32)
    o_ref[...] = h * dinv


def _tc_last_body(z0_ref, z1_ref, u_ref, d0_ref, d1_ref, b_ref, o_ref):
    dinv = _dinv_block(d0_ref, d1_ref)
    o_ref[...] = (z0_ref[...] + z1_ref[...] + u_ref[...]) * dinv + b_ref[...]


_spec_rows = pl.BlockSpec((_BM, F), lambda i: (i, 0))
_spec_deg = pl.BlockSpec((_BM, 16), lambda i: (i, 0))
_spec_w = pl.BlockSpec((F, F), lambda i: (0, 0))
_spec_b = pl.BlockSpec((1, F), lambda i: (0, 0))
_out_rows = jax.ShapeDtypeStruct((N, F), jnp.float32)


def _tc_first(x, W, deg0, deg1):
    return pl.pallas_call(
        _tc_first_body, grid=(N // _BM,),
        in_specs=[_spec_rows, _spec_w, _spec_deg, _spec_deg],
        out_specs=_spec_rows, out_shape=_out_rows,
    )(x, W, deg0, deg1)


def _tc_mid(z0, z1, u, deg0, deg1, b2d, W):
    return pl.pallas_call(
        _tc_mid_body, grid=(N // _BM,),
        in_specs=[_spec_rows, _spec_rows, _spec_rows, _spec_deg, _spec_deg,
                  _spec_b, _spec_w],
        out_specs=_spec_rows, out_shape=_out_rows,
    )(z0, z1, u, deg0, deg1, b2d, W)


def _tc_last(z0, z1, u, deg0, deg1, b2d):
    return pl.pallas_call(
        _tc_last_body, grid=(N // _BM,),
        in_specs=[_spec_rows, _spec_rows, _spec_rows, _spec_deg, _spec_deg,
                  _spec_b],
        out_specs=_spec_rows, out_shape=_out_rows,
    )(z0, z1, u, deg0, deg1, b2d)


def kernel(x, edge_index, W1, b1, W2, b2, W3, b3):
    ei = edge_index.astype(jnp.int32)
    pad = CAP_E - E
    rowp = jnp.concatenate([ei[0], jnp.full((pad,), N, jnp.int32)])
    colp = jnp.concatenate([ei[1], jnp.arange(pad, dtype=jnp.int32) % N])
    row2 = rowp.reshape(TOTCH, B)
    col2 = colp.reshape(TOTCH, B)
    pad_deg = NW * NCH * B - E
    col3 = jnp.concatenate(
        [ei[1], jnp.full((pad_deg,), N, jnp.int32)]).reshape(NW, NCH, B)

    onesb = jnp.ones((B, F), jnp.float32)
    zrows = jnp.zeros((RINIT, F), jnp.float32)

    zrow8 = jnp.zeros((8, F), jnp.float32)

    dd = _sc_degree(col3, onesb, zrows)[:, :, :16]
    deg0, deg1 = dd[0], dd[1]
    u = _tc_first(x, W1, deg0, deg1)
    z = _sc_agg(jnp.concatenate([u, zrow8]), row2, col2, zrows)
    u = _tc_mid(z[0], z[1], u, deg0, deg1, b1.reshape(1, F), W2)
    z = _sc_agg(jnp.concatenate([u, zrow8]), row2, col2, zrows)
    u = _tc_mid(z[0], z[1], u, deg0, deg1, b2.reshape(1, F), W3)
    z = _sc_agg(jnp.concatenate([u, zrow8]), row2, col2, zrows)
    return _tc_last(z[0], z[1], u, deg0, deg1, b3.reshape(1, F))


# speedup vs baseline: 2.3813x; 2.3813x over previous
import functools

import jax
import jax.numpy as jnp
from jax import lax
from jax.experimental import pallas as pl
from jax.experimental.pallas import tpu as pltpu
from jax.experimental.pallas import tpu_sc as plsc

N = 10000
F = 128
E = 320000
NC = 2
NS = 16
NW = NC * NS
B = 128
NCH = 80
EPW = B * NCH
EPAD = NW * EPW
NPAD = 10016
RINIT = 1000
RCOPY = 2000

SZ0 = (56, 56, 48)
SZ1 = (0, 0, 0)
K0 = sum(SZ0)
K1 = sum(SZ1)
KSTMAX = max(SZ0)
C1BASE = NS * K0
TOTCH = C1BASE
CAP_E = TOTCH * B

_mesh = plsc.VectorSubcoreMesh(core_axis_name="c", subcore_axis_name="s")


@functools.partial(
    pl.kernel,
    out_type=jax.ShapeDtypeStruct((NC, N, F), jnp.float32),
    mesh=_mesh,
    scratch_types=[
        pltpu.VMEM((NCH, B), jnp.int32),
        pltpu.VMEM((B, F), jnp.float32),
        pltpu.VMEM_SHARED((NPAD, F), jnp.float32),
    ],
)
def _sc_degree(col3, ones_hbm, zrows_hbm, dout, cidx, onesv, dacc):
    c = lax.axis_index("c")
    s = lax.axis_index("s")
    wid = s * NC + c
    pltpu.sync_copy(col3.at[wid], cidx)
    pltpu.sync_copy(ones_hbm, onesv)

    @pl.when(s < 10)
    def _():
        pltpu.sync_copy(zrows_hbm, dacc.at[pl.ds(s * RINIT, RINIT)])

    plsc.subcore_barrier()

    def body(j, carry):
        pltpu.sync_copy(onesv, dacc.at[cidx.at[j]], add=True)
        return carry

    lax.fori_loop(0, NCH, body, 0)
    plsc.subcore_barrier()

    @pl.when(s < 5)
    def _():
        pltpu.sync_copy(dacc.at[pl.ds(s * RCOPY, RCOPY)],
                        dout.at[c, pl.ds(s * RCOPY, RCOPY)])


@functools.partial(
    pl.kernel,
    out_type=jax.ShapeDtypeStruct((NC, N, F), jnp.float32),
    mesh=_mesh,
    scratch_types=[
        pltpu.VMEM((KSTMAX, B), jnp.int32),
        pltpu.VMEM((KSTMAX, B), jnp.int32),
        pltpu.VMEM((B, F), jnp.float32),
        pltpu.VMEM((B, F), jnp.float32),
        pltpu.VMEM_SHARED((NPAD, F), jnp.float32),
        pltpu.SemaphoreType.DMA,
        pltpu.SemaphoreType.DMA,
        pltpu.SemaphoreType.DMA,
        pltpu.SemaphoreType.DMA,
    ],
)
def _sc_agg(u_hbm, row2, col2, zrows_hbm, zout,
            ridx, cidx, rows0, rows1, acc, gs0, gs1, ss0, ss1):
    c = lax.axis_index("c")
    s = lax.axis_index("s")
    base = jnp.where(c == 0, s * K0, C1BASE + s * K1)

    @pl.when(s < 10)
    def _():
        pltpu.sync_copy(zrows_hbm, acc.at[pl.ds(s * RINIT, RINIT)])

    plsc.subcore_barrier()

    def _gather(j, rows, sem):
        return pltpu.async_copy(u_hbm.at[ridx.at[j]], rows, sem)

    def _gather_wait(j, rows, sem):
        pltpu.make_async_copy(u_hbm.at[ridx.at[j]], rows, sem).wait()

    def _scatter(j, rows, sem):
        return pltpu.async_copy(rows, acc.at[cidx.at[j]], sem, add=True)

    def _scatter_wait(j, rows, sem):
        pltpu.make_async_copy(rows, acc.at[cidx.at[j]], sem).wait()

    off0 = off1 = 0
    for st in range(len(SZ0)):
        start = pl.multiple_of(
            base + jnp.where(c == 0, off0, off1), 8)
        szc = jnp.where(c == 0, SZ0[st], SZ1[st])
        npair = jnp.where(c == 0, SZ0[st] // 2, SZ1[st] // 2)
        off0 += SZ0[st]
        off1 += SZ1[st]

        @pl.when(npair > 0)
        def _():
            pltpu.sync_copy(row2.at[pl.ds(start, KSTMAX)], ridx)
            pltpu.sync_copy(col2.at[pl.ds(start, KSTMAX)], cidx)
            _gather(0, rows0, gs0)
            _gather(1, rows1, gs1)

        def body(jp, carry):
            j0 = 2 * jp
            _gather_wait(j0, rows0, gs0)
            _scatter(j0, rows0, ss0)
            _gather_wait(j0 + 1, rows1, gs1)
            _scatter(j0 + 1, rows1, ss1)

            @pl.when(jp < npair - 1)
            def _():
                _scatter_wait(j0, rows0, ss0)
                _gather(j0 + 2, rows0, gs0)
                _scatter_wait(j0 + 1, rows1, ss1)
                _gather(j0 + 3, rows1, gs1)

            return carry

        lax.fori_loop(0, npair, body, 0)

        @pl.when(npair > 0)
        def _():
            _scatter_wait(szc - 2, rows0, ss0)
            _scatter_wait(szc - 1, rows1, ss1)

    plsc.subcore_barrier()

    @pl.when(s < 5)
    def _():
        pltpu.sync_copy(acc.at[pl.ds(s * RCOPY, RCOPY)],
                        zout.at[c, pl.ds(s * RCOPY, RCOPY)])


_BM = 2000


def _dinv_block(d0_ref, d1_ref):
    deg = d0_ref[:, 0:1] + d1_ref[:, 0:1] + 1.0
    return lax.rsqrt(deg)


def _tc_first_body(x_ref, w_ref, d0_ref, d1_ref, o_ref):
    h = jnp.dot(x_ref[...], w_ref[...], preferred_element_type=jnp.float32)
    o_ref[...] = h * _dinv_block(d0_ref, d1_ref)


def _tc_mid_body(z0_ref, z1_ref, u_ref, d0_ref, d1_ref, b_ref, w_ref, o_ref):
    dinv = _dinv_block(d0_ref, d1_ref)
    t = (z0_ref[...] + z1_ref[...] + u_ref[...]) * dinv + b_ref[...]
    t = jnp.maximum(t, 0.0)
    h = jnp.dot(t, w_ref[...], preferred_element_type=jnp.float32)
    o_ref[...] = h * dinv


def _tc_last_body(z0_ref, z1_ref, u_ref, d0_ref, d1_ref, b_ref, o_ref):
    dinv = _dinv_block(d0_ref, d1_ref)
    o_ref[...] = (z0_ref[...] + z1_ref[...] + u_ref[...]) * dinv + b_ref[...]


_spec_rows = pl.BlockSpec((_BM, F), lambda i: (i, 0))
_spec_deg = pl.BlockSpec((_BM, 16), lambda i: (i, 0))
_spec_w = pl.BlockSpec((F, F), lambda i: (0, 0))
_spec_b = pl.BlockSpec((1, F), lambda i: (0, 0))
_out_rows = jax.ShapeDtypeStruct((N, F), jnp.float32)


def _tc_first(x, W, deg0, deg1):
    return pl.pallas_call(
        _tc_first_body, grid=(N // _BM,),
        in_specs=[_spec_rows, _spec_w, _spec_deg, _spec_deg],
        out_specs=_spec_rows, out_shape=_out_rows,
    )(x, W, deg0, deg1)


def _tc_mid(z0, z1, u, deg0, deg1, b2d, W):
    return pl.pallas_call(
        _tc_mid_body, grid=(N // _BM,),
        in_specs=[_spec_rows, _spec_rows, _spec_rows, _spec_deg, _spec_deg,
                  _spec_b, _spec_w],
        out_specs=_spec_rows, out_shape=_out_rows,
    )(z0, z1, u, deg0, deg1, b2d, W)


def _tc_last(z0, z1, u, deg0, deg1, b2d):
    return pl.pallas_call(
        _tc_last_body, grid=(N // _BM,),
        in_specs=[_spec_rows, _spec_rows, _spec_rows, _spec_deg, _spec_deg,
                  _spec_b],
        out_specs=_spec_rows, out_shape=_out_rows,
    )(z0, z1, u, deg0, deg1, b2d)


def kernel(x, edge_index, W1, b1, W2, b2, W3, b3):
    ei = edge_index.astype(jnp.int32)
    pad = CAP_E - E
    arp = jnp.arange(pad, dtype=jnp.int32)
    rowp = jnp.concatenate([ei[0], arp % N])
    colp = jnp.concatenate([ei[1], N + (arp % (NPAD - N))])
    row2 = rowp.reshape(TOTCH, B)
    col2 = colp.reshape(TOTCH, B)
    pad_deg = NW * NCH * B - E
    col3 = jnp.concatenate(
        [ei[1], jnp.full((pad_deg,), N, jnp.int32)]).reshape(NW, NCH, B)

    onesb = jnp.ones((B, F), jnp.float32)
    zrows = jnp.zeros((RINIT, F), jnp.float32)

    dd = _sc_degree(col3, onesb, zrows)[:, :, :16]
    deg0, deg1 = dd[0], dd[1]
    u = _tc_first(x, W1, deg0, deg1)
    z = _sc_agg(u, row2, col2, zrows)
    u = _tc_mid(z[0], z[1], u, deg0, deg1, b1.reshape(1, F), W2)
    z = _sc_agg(u, row2, col2, zrows)
    u = _tc_mid(z[0], z[1], u, deg0, deg1, b2.reshape(1, F), W3)
    z = _sc_agg(u, row2, col2, zrows)
    return _tc_last(z[0], z[1], u, deg0, deg1, b3.reshape(1, F))


# speedup vs baseline: 3.8677x; 1.6242x over previous
import functools

import jax
import jax.numpy as jnp
from jax import lax
from jax.experimental import pallas as pl
from jax.experimental.pallas import tpu as pltpu
from jax.experimental.pallas import tpu_sc as plsc

N = 10000
F = 128
E = 320000
NC = 2
NS = 16
NW = NC * NS
B = 128
NCH = 80
EPW = B * NCH
EPAD = NW * EPW
NPAD = 10016
RINIT = 1000
RCOPY = 2000

SZ0 = (40, 40)
SZ1 = (40, 40)
K0 = sum(SZ0)
K1 = sum(SZ1)
KSTMAX = max(SZ0)
C1BASE = NS * K0
TOTCH = C1BASE + NS * K1
CAP_E = TOTCH * B

_mesh = plsc.VectorSubcoreMesh(core_axis_name="c", subcore_axis_name="s")


@functools.partial(
    pl.kernel,
    out_type=jax.ShapeDtypeStruct((NC, N, F), jnp.float32),
    mesh=_mesh,
    scratch_types=[
        pltpu.VMEM((NCH, B), jnp.int32),
        pltpu.VMEM((B, F), jnp.float32),
        pltpu.VMEM_SHARED((NPAD, F), jnp.float32),
    ],
)
def _sc_degree(col3, ones_hbm, zrows_hbm, dout, cidx, onesv, dacc):
    c = lax.axis_index("c")
    s = lax.axis_index("s")
    wid = s * NC + c
    pltpu.sync_copy(col3.at[wid], cidx)
    pltpu.sync_copy(ones_hbm, onesv)

    @pl.when(s < 10)
    def _():
        pltpu.sync_copy(zrows_hbm, dacc.at[pl.ds(s * RINIT, RINIT)])

    plsc.subcore_barrier()

    def body(j, carry):
        pltpu.sync_copy(onesv, dacc.at[cidx.at[j]], add=True)
        return carry

    lax.fori_loop(0, NCH, body, 0)
    plsc.subcore_barrier()

    @pl.when(s < 5)
    def _():
        pltpu.sync_copy(dacc.at[pl.ds(s * RCOPY, RCOPY)],
                        dout.at[c, pl.ds(s * RCOPY, RCOPY)])


@functools.partial(
    pl.kernel,
    out_type=jax.ShapeDtypeStruct((NC, N, F), jnp.float32),
    mesh=_mesh,
    scratch_types=[
        pltpu.VMEM((KSTMAX, B), jnp.int32),
        pltpu.VMEM((KSTMAX, B), jnp.int32),
        pltpu.VMEM((B, F), jnp.float32),
        pltpu.VMEM((B, F), jnp.float32),
        pltpu.VMEM_SHARED((NPAD, F), jnp.float32),
        pltpu.SemaphoreType.DMA,
        pltpu.SemaphoreType.DMA,
        pltpu.SemaphoreType.DMA,
        pltpu.SemaphoreType.DMA,
    ],
)
def _sc_agg(u_hbm, row2, col2, zrows_hbm, zout,
            ridx, cidx, rows0, rows1, acc, gs0, gs1, ss0, ss1):
    c = lax.axis_index("c")
    s = lax.axis_index("s")
    base = jnp.where(c == 0, s * K0, C1BASE + s * K1)

    @pl.when(s < 10)
    def _():
        pltpu.sync_copy(zrows_hbm, acc.at[pl.ds(s * RINIT, RINIT)])

    plsc.subcore_barrier()

    def _gather(j, rows, sem):
        return pltpu.async_copy(u_hbm.at[ridx.at[j]], rows, sem)

    def _gather_wait(j, rows, sem):
        pltpu.make_async_copy(u_hbm.at[ridx.at[j]], rows, sem).wait()

    def _scatter(j, rows, sem):
        return pltpu.async_copy(rows, acc.at[cidx.at[j]], sem, add=True)

    def _scatter_wait(j, rows, sem):
        pltpu.make_async_copy(rows, acc.at[cidx.at[j]], sem).wait()

    off0 = off1 = 0
    for st in range(len(SZ0)):
        start = pl.multiple_of(
            base + jnp.where(c == 0, off0, off1), 8)
        szc = jnp.where(c == 0, SZ0[st], SZ1[st])
        npair = jnp.where(c == 0, SZ0[st] // 2, SZ1[st] // 2)
        off0 += SZ0[st]
        off1 += SZ1[st]

        @pl.when(npair > 0)
        def _():
            pltpu.sync_copy(row2.at[pl.ds(start, KSTMAX)], ridx)
            pltpu.sync_copy(col2.at[pl.ds(start, KSTMAX)], cidx)
            _gather(0, rows0, gs0)
            _gather(1, rows1, gs1)

        def body(jp, carry):
            j0 = 2 * jp
            _gather_wait(j0, rows0, gs0)
            _scatter(j0, rows0, ss0)
            _gather_wait(j0 + 1, rows1, gs1)
            _scatter(j0 + 1, rows1, ss1)

            @pl.when(jp < npair - 1)
            def _():
                _scatter_wait(j0, rows0, ss0)
                _gather(j0 + 2, rows0, gs0)
                _scatter_wait(j0 + 1, rows1, ss1)
                _gather(j0 + 3, rows1, gs1)

            return carry

        lax.fori_loop(0, npair, body, 0)

        @pl.when(npair > 0)
        def _():
            _scatter_wait(szc - 2, rows0, ss0)
            _scatter_wait(szc - 1, rows1, ss1)

    plsc.subcore_barrier()

    @pl.when(s < 5)
    def _():
        pltpu.sync_copy(acc.at[pl.ds(s * RCOPY, RCOPY)],
                        zout.at[c, pl.ds(s * RCOPY, RCOPY)])


_BM = 2000


def _dinv_block(d0_ref, d1_ref):
    deg = d0_ref[:, 0:1] + d1_ref[:, 0:1] + 1.0
    return lax.rsqrt(deg)


def _tc_first_body(x_ref, w_ref, d0_ref, d1_ref, o_ref):
    h = jnp.dot(x_ref[...], w_ref[...], preferred_element_type=jnp.float32)
    o_ref[...] = h * _dinv_block(d0_ref, d1_ref)


def _tc_mid_body(z0_ref, z1_ref, u_ref, d0_ref, d1_ref, b_ref, w_ref, o_ref):
    dinv = _dinv_block(d0_ref, d1_ref)
    t = (z0_ref[...] + z1_ref[...] + u_ref[...]) * dinv + b_ref[...]
    t = jnp.maximum(t, 0.0)
    h = jnp.dot(t, w_ref[...], preferred_element_type=jnp.float32)
    o_ref[...] = h * dinv


def _tc_last_body(z0_ref, z1_ref, u_ref, d0_ref, d1_ref, b_ref, o_ref):
    dinv = _dinv_block(d0_ref, d1_ref)
    o_ref[...] = (z0_ref[...] + z1_ref[...] + u_ref[...]) * dinv + b_ref[...]


_spec_rows = pl.BlockSpec((_BM, F), lambda i: (i, 0))
_spec_deg = pl.BlockSpec((_BM, 16), lambda i: (i, 0))
_spec_w = pl.BlockSpec((F, F), lambda i: (0, 0))
_spec_b = pl.BlockSpec((1, F), lambda i: (0, 0))
_out_rows = jax.ShapeDtypeStruct((N, F), jnp.float32)


def _tc_first(x, W, deg0, deg1):
    return pl.pallas_call(
        _tc_first_body, grid=(N // _BM,),
        in_specs=[_spec_rows, _spec_w, _spec_deg, _spec_deg],
        out_specs=_spec_rows, out_shape=_out_rows,
    )(x, W, deg0, deg1)


def _tc_mid(z0, z1, u, deg0, deg1, b2d, W):
    return pl.pallas_call(
        _tc_mid_body, grid=(N // _BM,),
        in_specs=[_spec_rows, _spec_rows, _spec_rows, _spec_deg, _spec_deg,
                  _spec_b, _spec_w],
        out_specs=_spec_rows, out_shape=_out_rows,
    )(z0, z1, u, deg0, deg1, b2d, W)


def _tc_last(z0, z1, u, deg0, deg1, b2d):
    return pl.pallas_call(
        _tc_last_body, grid=(N // _BM,),
        in_specs=[_spec_rows, _spec_rows, _spec_rows, _spec_deg, _spec_deg,
                  _spec_b],
        out_specs=_spec_rows, out_shape=_out_rows,
    )(z0, z1, u, deg0, deg1, b2d)


def kernel(x, edge_index, W1, b1, W2, b2, W3, b3):
    ei = edge_index.astype(jnp.int32)
    pad = CAP_E - E
    arp = jnp.arange(pad, dtype=jnp.int32)
    rowp = jnp.concatenate([ei[0], arp % N])
    colp = jnp.concatenate([ei[1], N + (arp % (NPAD - N))])
    row2 = rowp.reshape(TOTCH, B)
    col2 = colp.reshape(TOTCH, B)
    pad_deg = NW * NCH * B - E
    ard = jnp.arange(pad_deg, dtype=jnp.int32)
    col3 = jnp.concatenate(
        [ei[1], N + (ard % (NPAD - N))]).reshape(NW, NCH, B)

    onesb = jnp.ones((B, F), jnp.float32)
    zrows = jnp.zeros((RINIT, F), jnp.float32)

    dd = _sc_degree(col3, onesb, zrows)[:, :, :16]
    deg0, deg1 = dd[0], dd[1]
    u = _tc_first(x, W1, deg0, deg1)
    z = _sc_agg(u, row2, col2, zrows)
    u = _tc_mid(z[0], z[1], u, deg0, deg1, b1.reshape(1, F), W2)
    z = _sc_agg(u, row2, col2, zrows)
    u = _tc_mid(z[0], z[1], u, deg0, deg1, b2.reshape(1, F), W3)
    z = _sc_agg(u, row2, col2, zrows)
    return _tc_last(z[0], z[1], u, deg0, deg1, b3.reshape(1, F))
